# baseline (speedup 1.0000x reference)
import functools

import jax
import jax.numpy as jnp
from jax import lax
from jax.experimental import pallas as pl
from jax.experimental.pallas import tpu as pltpu
from jax.experimental.pallas import tpu_sc as plsc

T, D, E, TK, I = 2048, 1024, 8, 2, 1024
BLK = 256
TB = T // BLK
NB_R = (T * TK) // BLK + E
NP_R = NB_R * BLK
NW = 32
GCHUNK = 48
CH4 = 32

_f32 = jnp.float32


def _router_body(h_ref, rw_ref, logits_ref, i12_ref, hs_ref):
    k = pl.program_id(0)
    x = h_ref[...]
    l = lax.dot_general(x, rw_ref[...], (((1,), (1,)), ((), ())),
                        preferred_element_type=_f32)
    iota_e = lax.broadcasted_iota(jnp.int32, (BLK, E), 1)
    m1 = jnp.max(l, axis=1, keepdims=True)
    i1 = jnp.min(jnp.where(l == m1, iota_e, E), axis=1, keepdims=True)
    l2 = jnp.where(iota_e == i1, -jnp.inf, l)
    m2 = jnp.max(l2, axis=1, keepdims=True)
    i2 = jnp.min(jnp.where(l2 == m2, iota_e, E), axis=1, keepdims=True)
    logits_ref[...] = l
    i12_ref[...] = jnp.where(iota_e == 0, i1, jnp.where(iota_e == 1, i2, 0))
    mv = jnp.where(k == 0, m1, m2)
    hs_ref[...] = (x * (1.0 / (1.0 + jnp.exp(-mv)))).astype(jnp.bfloat16)


def _router(h2, router_w):
    return pl.pallas_call(
        _router_body,
        grid=(TK, TB),
        in_specs=[
            pl.BlockSpec((BLK, D), lambda k, i: (i, 0)),
            pl.BlockSpec((E, D), lambda k, i: (0, 0)),
        ],
        out_specs=[
            pl.BlockSpec((BLK, E), lambda k, i: (i, 0)),
            pl.BlockSpec((BLK, E), lambda k, i: (i, 0)),
            pl.BlockSpec((BLK, D), lambda k, i: (k * TB + i, 0)),
        ],
        out_shape=[
            jax.ShapeDtypeStruct((T, E), _f32),
            jax.ShapeDtypeStruct((T, E), jnp.int32),
            jax.ShapeDtypeStruct((TK * T, D), jnp.bfloat16),
        ],
    )(h2, router_w)


def _metadata(i12):
    e_flat = jnp.concatenate([i12[:, 0], i12[:, 1]])
    oh = jax.nn.one_hot(e_flat, E, dtype=jnp.int32)
    counts = jnp.sum(oh, axis=0)
    ranks = jnp.cumsum(oh, axis=0) - oh
    padded = ((counts + BLK - 1) // BLK) * BLK
    cum_pad = jnp.cumsum(padded)
    starts = cum_pad - padded
    dest = starts[e_flat] + jnp.sum(ranks * oh, axis=1)
    pad_fill = jnp.arange(NP_R, dtype=jnp.int32) % (TK * T)
    flat_idx = pad_fill.at[dest].set(jnp.arange(TK * T, dtype=jnp.int32))
    pos0, pos1 = dest[:T], dest[T:]
    bid = jnp.arange(NB_R, dtype=jnp.int32)
    be = jnp.minimum(
        jnp.searchsorted(cum_pad, bid * BLK, side="right"), E - 1
    ).astype(jnp.int32)
    ba = (bid * BLK < starts[be] + counts[be]).astype(jnp.int32)
    return flat_idx, pos0, pos1, be, ba


@functools.cache
def _sc_mesh():
    return plsc.VectorSubcoreMesh(core_axis_name="c", subcore_axis_name="s")


def _pipelined_gather(table_hbm, jobs, bufs):
    n = len(jobs)
    copies = [None, None]
    for c in range(n):
        iv, rv, sm = bufs[c % 2]
        idx_src, _ = jobs[c]
        pltpu.sync_copy(idx_src, iv)
        copies[c % 2] = pltpu.async_copy(table_hbm.at[iv], rv, sm)
        if c > 0:
            _, out_dst = jobs[c - 1]
            copies[(c - 1) % 2].wait()
            pltpu.sync_copy(bufs[(c - 1) % 2][1], out_dst)
    copies[(n - 1) % 2].wait()
    pltpu.sync_copy(bufs[(n - 1) % 2][1], jobs[n - 1][1])


@functools.cache
def _sc_dispatch_kernel():
    @functools.partial(
        pl.kernel,
        mesh=_sc_mesh(),
        out_type=jax.ShapeDtypeStruct((NP_R, D // 2), jnp.int32),
        scratch_types=[
            pltpu.VMEM((GCHUNK,), jnp.int32),
            pltpu.VMEM((GCHUNK, D // 2), jnp.int32),
            pltpu.SemaphoreType.DMA,
            pltpu.VMEM((GCHUNK,), jnp.int32),
            pltpu.VMEM((GCHUNK, D // 2), jnp.int32),
            pltpu.SemaphoreType.DMA,
        ],
    )
    def body(hs_hbm, idx_hbm, out_hbm, ia, ra, sa, ib, rb, sb):
        w = lax.axis_index("s") * 2 + lax.axis_index("c")
        base = w * (NP_R // NW)
        jobs = []
        for c in range(NP_R // NW // GCHUNK):
            off = base + c * GCHUNK
            jobs.append((idx_hbm.at[pl.ds(off, GCHUNK)],
                         out_hbm.at[pl.ds(off, GCHUNK)]))
        _pipelined_gather(hs_hbm, jobs, ((ia, ra, sa), (ib, rb, sb)))

    return body


def _as_i32(x):
    n, d = x.shape
    return jax.lax.bitcast_convert_type(
        x.reshape(n, d // 2, 2), jnp.int32)


def _as_bf16(x):
    n, d2 = x.shape
    return jax.lax.bitcast_convert_type(x, jnp.bfloat16).reshape(n, 2 * d2)


def _sc_dispatch(hs, flat_idx):
    return _as_bf16(_sc_dispatch_kernel()(_as_i32(hs), flat_idx))


def _moe_mm_body(be_ref, ba_ref, x_ref, gw_ref, uw_ref, dw_ref, o_ref,
                 gwb, uwb, dwb):
    b = pl.program_id(0)
    bp = jnp.maximum(b - 1, 0)

    @pl.when(jnp.logical_or(b == 0, be_ref[b] != be_ref[bp]))
    def _():
        gwb[...] = gw_ref[0].astype(jnp.bfloat16)
        uwb[...] = uw_ref[0].astype(jnp.bfloat16)
        dwb[...] = dw_ref[0].astype(jnp.bfloat16)

    @pl.when(ba_ref[b] != 0)
    def _():
        x = x_ref[...]
        g = lax.dot_general(x, gwb[...], (((1,), (1,)), ((), ())),
                            preferred_element_type=_f32)
        u = lax.dot_general(x, uwb[...], (((1,), (1,)), ((), ())),
                            preferred_element_type=_f32)
        a = (g * (1.0 / (1.0 + jnp.exp(-g))) * u).astype(jnp.bfloat16)
        o_ref[...] = lax.dot_general(a, dwb[...], (((1,), (1,)), ((), ())),
                                     preferred_element_type=_f32
                                     ).astype(jnp.bfloat16)


def _moe_mm(be, ba, x_sorted, gate_w, up_w, down_w):
    grid_spec = pltpu.PrefetchScalarGridSpec(
        num_scalar_prefetch=2,
        grid=(NB_R,),
        in_specs=[
            pl.BlockSpec((BLK, D), lambda b, be, ba: (b, 0)),
            pl.BlockSpec((1, I, D), lambda b, be, ba: (be[b], 0, 0)),
            pl.BlockSpec((1, I, D), lambda b, be, ba: (be[b], 0, 0)),
            pl.BlockSpec((1, D, I), lambda b, be, ba: (be[b], 0, 0)),
        ],
        out_specs=pl.BlockSpec((BLK, D), lambda b, be, ba: (b, 0)),
        scratch_shapes=[
            pltpu.VMEM((I, D), jnp.bfloat16),
            pltpu.VMEM((I, D), jnp.bfloat16),
            pltpu.VMEM((D, I), jnp.bfloat16),
        ],
    )
    return pl.pallas_call(
        _moe_mm_body,
        grid_spec=grid_spec,
        out_shape=jax.ShapeDtypeStruct((NP_R, D), jnp.bfloat16),
    )(be, ba, x_sorted, gate_w, up_w, down_w)


@functools.cache
def _sc_return_kernel():
    @functools.partial(
        pl.kernel,
        mesh=_sc_mesh(),
        out_type=(
            jax.ShapeDtypeStruct((T, D // 2), jnp.int32),
            jax.ShapeDtypeStruct((T, D // 2), jnp.int32),
        ),
        scratch_types=[
            pltpu.VMEM((CH4,), jnp.int32),
            pltpu.VMEM((CH4, D // 2), jnp.int32),
            pltpu.SemaphoreType.DMA,
            pltpu.VMEM((CH4,), jnp.int32),
            pltpu.VMEM((CH4, D // 2), jnp.int32),
            pltpu.SemaphoreType.DMA,
        ],
    )
    def body(o_hbm, p0_hbm, p1_hbm, o0_hbm, o1_hbm, ia, ra, sa, ib, rb, sb):
        w = lax.axis_index("s") * 2 + lax.axis_index("c")
        base = w * (T // NW)
        jobs = []
        for src, dst in ((p0_hbm, o0_hbm), (p1_hbm, o1_hbm)):
            for c in range(T // NW // CH4):
                off = base + c * CH4
                jobs.append((src.at[pl.ds(off, CH4)],
                             dst.at[pl.ds(off, CH4)]))
        _pipelined_gather(o_hbm, jobs, ((ia, ra, sa), (ib, rb, sb)))

    return body


def _sc_return(o, pos0, pos1):
    o0, o1 = _sc_return_kernel()(_as_i32(o), pos0, pos1)
    return _as_bf16(o0), _as_bf16(o1)


def _shared_body(h_ref, gw_ref, uw_ref, dw_ref, o0_ref, o1_ref, out_ref,
                 gwb, uwb, dwb):
    @pl.when(pl.program_id(0) == 0)
    def _():
        gwb[...] = gw_ref[...].astype(jnp.bfloat16)
        uwb[...] = uw_ref[...].astype(jnp.bfloat16)
        dwb[...] = dw_ref[...].astype(jnp.bfloat16)

    x = h_ref[...].astype(jnp.bfloat16)
    g = lax.dot_general(x, gwb[...], (((1,), (1,)), ((), ())),
                        preferred_element_type=_f32)
    u = lax.dot_general(x, uwb[...], (((1,), (1,)), ((), ())),
                        preferred_element_type=_f32)
    a = (g * (1.0 / (1.0 + jnp.exp(-g))) * u).astype(jnp.bfloat16)
    sh = lax.dot_general(a, dwb[...], (((1,), (1,)), ((), ())),
                         preferred_element_type=_f32)
    out_ref[...] = sh + o0_ref[...].astype(_f32) + o1_ref[...].astype(_f32)


def _shared_combine(h2, sgw, suw, sdw, o0, o1):
    return pl.pallas_call(
        _shared_body,
        grid=(TB,),
        in_specs=[
            pl.BlockSpec((BLK, D), lambda i: (i, 0)),
            pl.BlockSpec((I, D), lambda i: (0, 0)),
            pl.BlockSpec((I, D), lambda i: (0, 0)),
            pl.BlockSpec((D, I), lambda i: (0, 0)),
            pl.BlockSpec((BLK, D), lambda i: (i, 0)),
            pl.BlockSpec((BLK, D), lambda i: (i, 0)),
        ],
        out_specs=pl.BlockSpec((BLK, D), lambda i: (i, 0)),
        out_shape=jax.ShapeDtypeStruct((T, D), _f32),
        scratch_shapes=[
            pltpu.VMEM((I, D), jnp.bfloat16),
            pltpu.VMEM((I, D), jnp.bfloat16),
            pltpu.VMEM((D, I), jnp.bfloat16),
        ],
    )(h2, sgw, suw, sdw, o0, o1)


def kernel(hidden_states, router_w, gate_w, up_w, down_w,
           shared_gate_w, shared_up_w, shared_down_w):
    b, s, d = hidden_states.shape
    h2 = hidden_states.reshape(T, D)
    logits, i12, hs = _router(h2, router_w)
    flat_idx, pos0, pos1, be, ba = _metadata(i12)
    x_sorted = _sc_dispatch(hs, flat_idx)
    o = _moe_mm(be, ba, x_sorted, gate_w, up_w, down_w)
    o0, o1 = _sc_return(o, pos0, pos1)
    out = _shared_combine(h2, shared_gate_w, shared_up_w, shared_down_w, o0, o1)
    return out.reshape(b, s, d), logits.reshape(b, s, E)


# speedup vs baseline: 3.0816x; 3.0816x over previous
import functools

import jax
import jax.numpy as jnp
from jax import lax
from jax.experimental import pallas as pl
from jax.experimental.pallas import tpu as pltpu
from jax.experimental.pallas import tpu_sc as plsc

T, D, E, TK, I = 2048, 1024, 8, 2, 1024
BLK = 256
TB = T // BLK
NB_R = (T * TK) // BLK + E
NP_R = NB_R * BLK
NW = 32
GCHUNK = 48
CH4 = 32

_f32 = jnp.float32


def _router_body(h_ref, rw_ref, logits_ref, i12_ref, hs_ref):
    k = pl.program_id(0)
    x = h_ref[...]
    l = lax.dot_general(x, rw_ref[...], (((1,), (1,)), ((), ())),
                        preferred_element_type=_f32)
    iota_e = lax.broadcasted_iota(jnp.int32, (BLK, E), 1)
    m1 = jnp.max(l, axis=1, keepdims=True)
    i1 = jnp.min(jnp.where(l == m1, iota_e, E), axis=1, keepdims=True)
    l2 = jnp.where(iota_e == i1, -jnp.inf, l)
    m2 = jnp.max(l2, axis=1, keepdims=True)
    i2 = jnp.min(jnp.where(l2 == m2, iota_e, E), axis=1, keepdims=True)
    logits_ref[...] = l
    i12_ref[...] = jnp.where(iota_e == 0, i1, jnp.where(iota_e == 1, i2, 0))
    mv = jnp.where(k == 0, m1, m2)
    hs_ref[...] = x * (1.0 / (1.0 + jnp.exp(-mv)))


def _router(h2, router_w):
    return pl.pallas_call(
        _router_body,
        grid=(TK, TB),
        in_specs=[
            pl.BlockSpec((BLK, D), lambda k, i: (i, 0)),
            pl.BlockSpec((E, D), lambda k, i: (0, 0)),
        ],
        out_specs=[
            pl.BlockSpec((BLK, E), lambda k, i: (i, 0)),
            pl.BlockSpec((BLK, E), lambda k, i: (i, 0)),
            pl.BlockSpec((BLK, D), lambda k, i: (k * TB + i, 0)),
        ],
        out_shape=[
            jax.ShapeDtypeStruct((T, E), _f32),
            jax.ShapeDtypeStruct((T, E), jnp.int32),
            jax.ShapeDtypeStruct((TK * T, D), _f32),
        ],
    )(h2, router_w)


def _metadata(i12):
    e_flat = jnp.concatenate([i12[:, 0], i12[:, 1]])
    oh = jax.nn.one_hot(e_flat, E, dtype=jnp.int32)
    counts = jnp.sum(oh, axis=0)
    ranks = jnp.cumsum(oh, axis=0) - oh
    padded = ((counts + BLK - 1) // BLK) * BLK
    cum_pad = jnp.cumsum(padded)
    starts = cum_pad - padded
    dest = starts[e_flat] + jnp.sum(ranks * oh, axis=1)
    pad_fill = jnp.arange(NP_R, dtype=jnp.int32) % (TK * T)
    flat_idx = pad_fill.at[dest].set(jnp.arange(TK * T, dtype=jnp.int32))
    pos0, pos1 = dest[:T], dest[T:]
    bid = jnp.arange(NB_R, dtype=jnp.int32)
    be = jnp.minimum(
        jnp.searchsorted(cum_pad, bid * BLK, side="right"), E - 1
    ).astype(jnp.int32)
    ba = (bid * BLK < starts[be] + counts[be]).astype(jnp.int32)
    return flat_idx, pos0, pos1, be, ba


@functools.cache
def _sc_mesh():
    return plsc.VectorSubcoreMesh(core_axis_name="c", subcore_axis_name="s")


def _pipelined_gather(table_hbm, jobs, bufs):
    n = len(jobs)
    copies = [None, None]
    for c in range(n):
        iv, rv, sm = bufs[c % 2]
        idx_src, _ = jobs[c]
        pltpu.sync_copy(idx_src, iv)
        copies[c % 2] = pltpu.async_copy(table_hbm.at[iv], rv, sm)
        if c > 0:
            _, out_dst = jobs[c - 1]
            copies[(c - 1) % 2].wait()
            pltpu.sync_copy(bufs[(c - 1) % 2][1], out_dst)
    copies[(n - 1) % 2].wait()
    pltpu.sync_copy(bufs[(n - 1) % 2][1], jobs[n - 1][1])


@functools.cache
def _sc_dispatch_kernel():
    @functools.partial(
        pl.kernel,
        mesh=_sc_mesh(),
        out_type=jax.ShapeDtypeStruct((NP_R, D), _f32),
        scratch_types=[
            pltpu.VMEM((GCHUNK,), jnp.int32),
            pltpu.VMEM((GCHUNK, D), _f32),
            pltpu.SemaphoreType.DMA,
            pltpu.VMEM((GCHUNK,), jnp.int32),
            pltpu.VMEM((GCHUNK, D), _f32),
            pltpu.SemaphoreType.DMA,
        ],
    )
    def body(hs_hbm, idx_hbm, out_hbm, ia, ra, sa, ib, rb, sb):
        w = lax.axis_index("s") * 2 + lax.axis_index("c")
        base = w * (NP_R // NW)
        jobs = []
        for c in range(NP_R // NW // GCHUNK):
            off = base + c * GCHUNK
            jobs.append((idx_hbm.at[pl.ds(off, GCHUNK)],
                         out_hbm.at[pl.ds(off, GCHUNK)]))
        _pipelined_gather(hs_hbm, jobs, ((ia, ra, sa), (ib, rb, sb)))

    return body


def _sc_dispatch(hs, flat_idx):
    return _sc_dispatch_kernel()(hs, flat_idx)


def _moe_mm_body(be_ref, ba_ref, x_ref, gw_ref, uw_ref, dw_ref, o_ref,
                 gwb, uwb, dwb):
    b = pl.program_id(0)
    bp = jnp.maximum(b - 1, 0)

    @pl.when(jnp.logical_or(b == 0, be_ref[b] != be_ref[bp]))
    def _():
        gwb[...] = gw_ref[0].astype(jnp.bfloat16)
        uwb[...] = uw_ref[0].astype(jnp.bfloat16)
        dwb[...] = dw_ref[0].astype(jnp.bfloat16)

    @pl.when(ba_ref[b] != 0)
    def _():
        x = x_ref[...].astype(jnp.bfloat16)
        g = lax.dot_general(x, gwb[...], (((1,), (1,)), ((), ())),
                            preferred_element_type=_f32)
        u = lax.dot_general(x, uwb[...], (((1,), (1,)), ((), ())),
                            preferred_element_type=_f32)
        a = (g * (1.0 / (1.0 + jnp.exp(-g))) * u).astype(jnp.bfloat16)
        o_ref[...] = lax.dot_general(a, dwb[...], (((1,), (1,)), ((), ())),
                                     preferred_element_type=_f32)


def _moe_mm(be, ba, x_sorted, gate_w, up_w, down_w):
    grid_spec = pltpu.PrefetchScalarGridSpec(
        num_scalar_prefetch=2,
        grid=(NB_R,),
        in_specs=[
            pl.BlockSpec((BLK, D), lambda b, be, ba: (b, 0)),
            pl.BlockSpec((1, I, D), lambda b, be, ba: (be[b], 0, 0)),
            pl.BlockSpec((1, I, D), lambda b, be, ba: (be[b], 0, 0)),
            pl.BlockSpec((1, D, I), lambda b, be, ba: (be[b], 0, 0)),
        ],
        out_specs=pl.BlockSpec((BLK, D), lambda b, be, ba: (b, 0)),
        scratch_shapes=[
            pltpu.VMEM((I, D), jnp.bfloat16),
            pltpu.VMEM((I, D), jnp.bfloat16),
            pltpu.VMEM((D, I), jnp.bfloat16),
        ],
    )
    return pl.pallas_call(
        _moe_mm_body,
        grid_spec=grid_spec,
        out_shape=jax.ShapeDtypeStruct((NP_R, D), _f32),
    )(be, ba, x_sorted, gate_w, up_w, down_w)


@functools.cache
def _sc_return_kernel():
    @functools.partial(
        pl.kernel,
        mesh=_sc_mesh(),
        out_type=(
            jax.ShapeDtypeStruct((T, D), _f32),
            jax.ShapeDtypeStruct((T, D), _f32),
        ),
        scratch_types=[
            pltpu.VMEM((CH4,), jnp.int32),
            pltpu.VMEM((CH4, D), _f32),
            pltpu.SemaphoreType.DMA,
            pltpu.VMEM((CH4,), jnp.int32),
            pltpu.VMEM((CH4, D), _f32),
            pltpu.SemaphoreType.DMA,
        ],
    )
    def body(o_hbm, p0_hbm, p1_hbm, o0_hbm, o1_hbm, ia, ra, sa, ib, rb, sb):
        w = lax.axis_index("s") * 2 + lax.axis_index("c")
        base = w * (T // NW)
        jobs = []
        for src, dst in ((p0_hbm, o0_hbm), (p1_hbm, o1_hbm)):
            for c in range(T // NW // CH4):
                off = base + c * CH4
                jobs.append((src.at[pl.ds(off, CH4)],
                             dst.at[pl.ds(off, CH4)]))
        _pipelined_gather(o_hbm, jobs, ((ia, ra, sa), (ib, rb, sb)))

    return body


def _sc_return(o, pos0, pos1):
    return _sc_return_kernel()(o, pos0, pos1)


def _shared_body(h_ref, gw_ref, uw_ref, dw_ref, o0_ref, o1_ref, out_ref,
                 gwb, uwb, dwb):
    @pl.when(pl.program_id(0) == 0)
    def _():
        gwb[...] = gw_ref[...].astype(jnp.bfloat16)
        uwb[...] = uw_ref[...].astype(jnp.bfloat16)
        dwb[...] = dw_ref[...].astype(jnp.bfloat16)

    x = h_ref[...].astype(jnp.bfloat16)
    g = lax.dot_general(x, gwb[...], (((1,), (1,)), ((), ())),
                        preferred_element_type=_f32)
    u = lax.dot_general(x, uwb[...], (((1,), (1,)), ((), ())),
                        preferred_element_type=_f32)
    a = (g * (1.0 / (1.0 + jnp.exp(-g))) * u).astype(jnp.bfloat16)
    sh = lax.dot_general(a, dwb[...], (((1,), (1,)), ((), ())),
                         preferred_element_type=_f32)
    out_ref[...] = sh + o0_ref[...] + o1_ref[...]


def _shared_combine(h2, sgw, suw, sdw, o0, o1):
    return pl.pallas_call(
        _shared_body,
        grid=(TB,),
        in_specs=[
            pl.BlockSpec((BLK, D), lambda i: (i, 0)),
            pl.BlockSpec((I, D), lambda i: (0, 0)),
            pl.BlockSpec((I, D), lambda i: (0, 0)),
            pl.BlockSpec((D, I), lambda i: (0, 0)),
            pl.BlockSpec((BLK, D), lambda i: (i, 0)),
            pl.BlockSpec((BLK, D), lambda i: (i, 0)),
        ],
        out_specs=pl.BlockSpec((BLK, D), lambda i: (i, 0)),
        out_shape=jax.ShapeDtypeStruct((T, D), _f32),
        scratch_shapes=[
            pltpu.VMEM((I, D), jnp.bfloat16),
            pltpu.VMEM((I, D), jnp.bfloat16),
            pltpu.VMEM((D, I), jnp.bfloat16),
        ],
    )(h2, sgw, suw, sdw, o0, o1)


def kernel(hidden_states, router_w, gate_w, up_w, down_w,
           shared_gate_w, shared_up_w, shared_down_w):
    b, s, d = hidden_states.shape
    h2 = hidden_states.reshape(T, D)
    logits, i12, hs = _router(h2, router_w)
    flat_idx, pos0, pos1, be, ba = _metadata(i12)
    x_sorted = _sc_dispatch(hs, flat_idx)
    o = _moe_mm(be, ba, x_sorted, gate_w, up_w, down_w)
    o0, o1 = _sc_return(o, pos0, pos1)
    out = _shared_combine(h2, shared_gate_w, shared_up_w, shared_down_w, o0, o1)
    return out.reshape(b, s, d), logits.reshape(b, s, E)


# speedup vs baseline: 3.5649x; 1.1568x over previous
import functools

import jax
import jax.numpy as jnp
from jax import lax
from jax.experimental import pallas as pl
from jax.experimental.pallas import tpu as pltpu
from jax.experimental.pallas import tpu_sc as plsc

T, D, E, TK, I = 2048, 1024, 8, 2, 1024
BLK = 256
TB = T // BLK
NB_R = (T * TK) // BLK + E
NP_R = NB_R * BLK
NW = 32
DCH = 32
CH4 = 32

_f32 = jnp.float32


def _router_body(h_ref, rw_ref, logits_ref, i12_ref, hs_ref, rank_ref,
                 cnt_ref, run_ref):
    i = pl.program_id(0)
    x = h_ref[...]
    l = lax.dot_general(x, rw_ref[...], (((1,), (1,)), ((), ())),
                        preferred_element_type=_f32)
    iota_e = lax.broadcasted_iota(jnp.int32, (BLK, E), 1)
    m1 = jnp.max(l, axis=1, keepdims=True)
    i1 = jnp.min(jnp.where(l == m1, iota_e, E), axis=1, keepdims=True)
    l2 = jnp.where(iota_e == i1, -jnp.inf, l)
    m2 = jnp.max(l2, axis=1, keepdims=True)
    i2 = jnp.min(jnp.where(l2 == m2, iota_e, E), axis=1, keepdims=True)
    logits_ref[...] = l
    i12_ref[...] = jnp.where(iota_e == 0, i1, jnp.where(iota_e == 1, i2, 0))
    hs_ref[0] = x * (1.0 / (1.0 + jnp.exp(-m1)))
    hs_ref[1] = x * (1.0 / (1.0 + jnp.exp(-m2)))

    oh0 = (iota_e == i1).astype(_f32)
    oh1 = (iota_e == i2).astype(_f32)
    r_iota = lax.broadcasted_iota(jnp.int32, (BLK, BLK), 0)
    c_iota = lax.broadcasted_iota(jnp.int32, (BLK, BLK), 1)
    tri = (r_iota > c_iota).astype(_f32)
    cum0 = lax.dot_general(tri, oh0, (((1,), (0,)), ((), ())),
                           preferred_element_type=_f32)
    cum1 = lax.dot_general(tri, oh1, (((1,), (0,)), ((), ())),
                           preferred_element_type=_f32)
    col0 = jnp.sum(oh0, axis=0, keepdims=True)
    col1 = jnp.sum(oh1, axis=0, keepdims=True)
    run = jnp.where(i == 0, jnp.zeros((1, E), _f32), run_ref[...])
    r0 = jnp.sum(oh0 * (run + cum0), axis=1, keepdims=True)
    r1 = jnp.sum(oh1 * (run + col0 + cum1), axis=1, keepdims=True)
    rank_ref[0] = jnp.where(iota_e == 0, r0.astype(jnp.int32), 0)
    rank_ref[1] = jnp.where(iota_e == 0, r1.astype(jnp.int32), 0)
    new_run = run + col0 + col1
    run_ref[...] = new_run

    @pl.when(i == TB - 1)
    def _():
        cnt_ref[...] = new_run.astype(jnp.int32)


def _router(h2, router_w):
    return pl.pallas_call(
        _router_body,
        grid=(TB,),
        in_specs=[
            pl.BlockSpec((BLK, D), lambda i: (i, 0)),
            pl.BlockSpec((E, D), lambda i: (0, 0)),
        ],
        out_specs=[
            pl.BlockSpec((BLK, E), lambda i: (i, 0)),
            pl.BlockSpec((BLK, E), lambda i: (i, 0)),
            pl.BlockSpec((TK, BLK, D), lambda i: (0, i, 0)),
            pl.BlockSpec((TK, BLK, E), lambda i: (0, i, 0)),
            pl.BlockSpec((1, E), lambda i: (0, 0)),
        ],
        out_shape=[
            jax.ShapeDtypeStruct((T, E), _f32),
            jax.ShapeDtypeStruct((T, E), jnp.int32),
            jax.ShapeDtypeStruct((TK, T, D), _f32),
            jax.ShapeDtypeStruct((TK, T, E), jnp.int32),
            jax.ShapeDtypeStruct((1, E), jnp.int32),
        ],
        scratch_shapes=[pltpu.VMEM((1, E), _f32)],
    )(h2, router_w)


def _metadata(i12, rank01, cnt):
    e_flat = jnp.concatenate([i12[:, 0], i12[:, 1]])
    rank = rank01[:, :, 0].reshape(TK * T)
    counts = cnt[0]
    padded = ((counts + BLK - 1) // BLK) * BLK
    cum_pad = jnp.cumsum(padded)
    starts = cum_pad - padded
    dest = starts[e_flat] + rank
    pos0, pos1 = dest[:T], dest[T:]
    bid = jnp.arange(NB_R, dtype=jnp.int32)
    be = jnp.minimum(
        jnp.searchsorted(cum_pad, bid * BLK, side="right"), E - 1
    ).astype(jnp.int32)
    ba = (bid * BLK < starts[be] + counts[be]).astype(jnp.int32)
    return dest, pos0, pos1, be, ba


@functools.cache
def _sc_mesh():
    return plsc.VectorSubcoreMesh(core_axis_name="c", subcore_axis_name="s")


@functools.cache
def _sc_dispatch_kernel():
    @functools.partial(
        pl.kernel,
        mesh=_sc_mesh(),
        out_type=jax.ShapeDtypeStruct((NP_R, D), _f32),
        scratch_types=[
            pltpu.VMEM((DCH,), jnp.int32),
            pltpu.VMEM((DCH, D), _f32),
            pltpu.SemaphoreType.DMA,
            pltpu.VMEM((DCH,), jnp.int32),
            pltpu.VMEM((DCH, D), _f32),
            pltpu.SemaphoreType.DMA,
        ],
    )
    def body(hs_hbm, dest_hbm, out_hbm, ia, ra, sa, ib, rb, sb):
        w = lax.axis_index("s") * 2 + lax.axis_index("c")
        rows_per_w = TK * T // NW
        base = w * rows_per_w
        bufs = ((ia, ra, sa), (ib, rb, sb))
        nch = rows_per_w // DCH
        loads = [None, None]
        for c in range(nch):
            iv, rv, sm = bufs[c % 2]
            off = base + c * DCH
            pltpu.sync_copy(dest_hbm.at[pl.ds(off, DCH)], iv)
            loads[c % 2] = pltpu.async_copy(hs_hbm.at[pl.ds(off, DCH)], rv, sm)
            if c > 0:
                pi, pr, ps = bufs[(c - 1) % 2]
                loads[(c - 1) % 2].wait()
                pltpu.async_copy(pr, out_hbm.at[pi], ps).wait()
        loads[(nch - 1) % 2].wait()
        iv, rv, sm = bufs[(nch - 1) % 2]
        pltpu.async_copy(rv, out_hbm.at[iv], sm).wait()

    return body


def _sc_dispatch(hs, dest):
    return _sc_dispatch_kernel()(hs.reshape(TK * T, D), dest)


def _moe_mm_body(be_ref, ba_ref, x_ref, gw_ref, uw_ref, dw_ref, o_ref,
                 gwb, uwb, dwb):
    b = pl.program_id(0)
    bp = jnp.maximum(b - 1, 0)

    @pl.when(jnp.logical_or(b == 0, be_ref[b] != be_ref[bp]))
    def _():
        gwb[...] = gw_ref[0].astype(jnp.bfloat16)
        uwb[...] = uw_ref[0].astype(jnp.bfloat16)
        dwb[...] = dw_ref[0].astype(jnp.bfloat16)

    @pl.when(ba_ref[b] != 0)
    def _():
        x = x_ref[...].astype(jnp.bfloat16)
        g = lax.dot_general(x, gwb[...], (((1,), (1,)), ((), ())),
                            preferred_element_type=_f32)
        u = lax.dot_general(x, uwb[...], (((1,), (1,)), ((), ())),
                            preferred_element_type=_f32)
        a = (g * (1.0 / (1.0 + jnp.exp(-g))) * u).astype(jnp.bfloat16)
        o_ref[...] = lax.dot_general(a, dwb[...], (((1,), (1,)), ((), ())),
                                     preferred_element_type=_f32)


def _moe_mm(be, ba, x_sorted, gate_w, up_w, down_w):
    grid_spec = pltpu.PrefetchScalarGridSpec(
        num_scalar_prefetch=2,
        grid=(NB_R,),
        in_specs=[
            pl.BlockSpec((BLK, D), lambda b, be, ba: (b, 0)),
            pl.BlockSpec((1, I, D), lambda b, be, ba: (be[b], 0, 0)),
            pl.BlockSpec((1, I, D), lambda b, be, ba: (be[b], 0, 0)),
            pl.BlockSpec((1, D, I), lambda b, be, ba: (be[b], 0, 0)),
        ],
        out_specs=pl.BlockSpec((BLK, D), lambda b, be, ba: (b, 0)),
        scratch_shapes=[
            pltpu.VMEM((I, D), jnp.bfloat16),
            pltpu.VMEM((I, D), jnp.bfloat16),
            pltpu.VMEM((D, I), jnp.bfloat16),
        ],
    )
    return pl.pallas_call(
        _moe_mm_body,
        grid_spec=grid_spec,
        out_shape=jax.ShapeDtypeStruct((NP_R, D), _f32),
    )(be, ba, x_sorted, gate_w, up_w, down_w)


def _pipelined_gather(table_hbm, jobs, bufs):
    n = len(jobs)
    copies = [None, None]
    for c in range(n):
        iv, rv, sm = bufs[c % 2]
        idx_src, _ = jobs[c]
        pltpu.sync_copy(idx_src, iv)
        copies[c % 2] = pltpu.async_copy(table_hbm.at[iv], rv, sm)
        if c > 0:
            _, out_dst = jobs[c - 1]
            copies[(c - 1) % 2].wait()
            pltpu.sync_copy(bufs[(c - 1) % 2][1], out_dst)
    copies[(n - 1) % 2].wait()
    pltpu.sync_copy(bufs[(n - 1) % 2][1], jobs[n - 1][1])


@functools.cache
def _sc_return_kernel():
    @functools.partial(
        pl.kernel,
        mesh=_sc_mesh(),
        out_type=(
            jax.ShapeDtypeStruct((T, D), _f32),
            jax.ShapeDtypeStruct((T, D), _f32),
        ),
        scratch_types=[
            pltpu.VMEM((CH4,), jnp.int32),
            pltpu.VMEM((CH4, D), _f32),
            pltpu.SemaphoreType.DMA,
            pltpu.VMEM((CH4,), jnp.int32),
            pltpu.VMEM((CH4, D), _f32),
            pltpu.SemaphoreType.DMA,
        ],
    )
    def body(o_hbm, p0_hbm, p1_hbm, o0_hbm, o1_hbm, ia, ra, sa, ib, rb, sb):
        w = lax.axis_index("s") * 2 + lax.axis_index("c")
        base = w * (T // NW)
        jobs = []
        for src, dst in ((p0_hbm, o0_hbm), (p1_hbm, o1_hbm)):
            for c in range(T // NW // CH4):
                off = base + c * CH4
                jobs.append((src.at[pl.ds(off, CH4)],
                             dst.at[pl.ds(off, CH4)]))
        _pipelined_gather(o_hbm, jobs, ((ia, ra, sa), (ib, rb, sb)))

    return body


def _sc_return(o, pos0, pos1):
    return _sc_return_kernel()(o, pos0, pos1)


def _shared_body(h_ref, gw_ref, uw_ref, dw_ref, o0_ref, o1_ref, out_ref,
                 gwb, uwb, dwb):
    @pl.when(pl.program_id(0) == 0)
    def _():
        gwb[...] = gw_ref[...].astype(jnp.bfloat16)
        uwb[...] = uw_ref[...].astype(jnp.bfloat16)
        dwb[...] = dw_ref[...].astype(jnp.bfloat16)

    x = h_ref[...].astype(jnp.bfloat16)
    g = lax.dot_general(x, gwb[...], (((1,), (1,)), ((), ())),
                        preferred_element_type=_f32)
    u = lax.dot_general(x, uwb[...], (((1,), (1,)), ((), ())),
                        preferred_element_type=_f32)
    a = (g * (1.0 / (1.0 + jnp.exp(-g))) * u).astype(jnp.bfloat16)
    sh = lax.dot_general(a, dwb[...], (((1,), (1,)), ((), ())),
                         preferred_element_type=_f32)
    out_ref[...] = sh + o0_ref[...] + o1_ref[...]


def _shared_combine(h2, sgw, suw, sdw, o0, o1):
    return pl.pallas_call(
        _shared_body,
        grid=(TB,),
        in_specs=[
            pl.BlockSpec((BLK, D), lambda i: (i, 0)),
            pl.BlockSpec((I, D), lambda i: (0, 0)),
            pl.BlockSpec((I, D), lambda i: (0, 0)),
            pl.BlockSpec((D, I), lambda i: (0, 0)),
            pl.BlockSpec((BLK, D), lambda i: (i, 0)),
            pl.BlockSpec((BLK, D), lambda i: (i, 0)),
        ],
        out_specs=pl.BlockSpec((BLK, D), lambda i: (i, 0)),
        out_shape=jax.ShapeDtypeStruct((T, D), _f32),
        scratch_shapes=[
            pltpu.VMEM((I, D), jnp.bfloat16),
            pltpu.VMEM((I, D), jnp.bfloat16),
            pltpu.VMEM((D, I), jnp.bfloat16),
        ],
    )(h2, sgw, suw, sdw, o0, o1)


def kernel(hidden_states, router_w, gate_w, up_w, down_w,
           shared_gate_w, shared_up_w, shared_down_w):
    b, s, d = hidden_states.shape
    h2 = hidden_states.reshape(T, D)
    logits, i12, hs, rank01, cnt = _router(h2, router_w)
    dest, pos0, pos1, be, ba = _metadata(i12, rank01, cnt)
    x_sorted = _sc_dispatch(hs, dest)
    o = _moe_mm(be, ba, x_sorted, gate_w, up_w, down_w)
    o0, o1 = _sc_return(o, pos0, pos1)
    out = _shared_combine(h2, shared_gate_w, shared_up_w, shared_down_w, o0, o1)
    return out.reshape(b, s, d), logits.reshape(b, s, E)
